# baseline (speedup 1.0000x reference)
import jax
import jax.numpy as jnp
from jax import lax
from jax.experimental import pallas as pl
from jax.experimental.pallas import tpu as pltpu
from jax.experimental.pallas import tpu_sc as plsc

NC, NS = 2, 16
NW = NC * NS
B = 16384
BPW = B // NW
CH = 128
NCH = BPW // CH
WD, RD = 64, 16
OD = WD + RD + 1


def _body(weapon_hbm, rank_hbm, level_hbm, wtab_hbm, rtab_hbm, out_hbm,
          idx_w, idx_r, wrows, rrows, sem):
    wid = lax.axis_index("s") * NC + lax.axis_index("c")
    base = wid * BPW
    rows_out = pl.ds(base, BPW)
    pltpu.sync_copy(weapon_hbm.at[rows_out], idx_w)
    pltpu.sync_copy(rank_hbm.at[rows_out], idx_r)
    cps = []
    for j in range(NCH):
        r = pl.ds(j * CH, CH)
        cps.append(pltpu.async_copy(
            wtab_hbm.at[idx_w.at[r]], wrows.at[r], sem))
        cps.append(pltpu.async_copy(
            rtab_hbm.at[idx_r.at[r]], rrows.at[r], sem))
    pltpu.sync_copy(level_hbm.at[rows_out],
                    out_hbm.at[rows_out, pl.ds(WD + RD, 1)])
    for cp in cps:
        cp.wait()
    pltpu.sync_copy(wrows, out_hbm.at[rows_out, pl.ds(0, WD)])
    pltpu.sync_copy(rrows, out_hbm.at[rows_out, pl.ds(WD, RD)])


def kernel(weapon, rank, level, weapon_table, rank_table):
    mesh = plsc.VectorSubcoreMesh(core_axis_name="c", subcore_axis_name="s")
    k = pl.kernel(
        _body,
        out_type=jax.ShapeDtypeStruct((B, OD), jnp.float32),
        mesh=mesh,
        scratch_types=[
            pltpu.VMEM((BPW,), jnp.int32),
            pltpu.VMEM((BPW,), jnp.int32),
            pltpu.VMEM((BPW, WD), jnp.float32),
            pltpu.VMEM((BPW, RD), jnp.float32),
            pltpu.SemaphoreType.DMA,
        ],
        compiler_params=pltpu.CompilerParams(
            use_tc_tiling_on_sc=False, needs_layout_passes=False),
    )
    return k(weapon, rank, level[:, None], weapon_table, rank_table)


# speedup vs baseline: 1.0451x; 1.0451x over previous
import jax
import jax.numpy as jnp
from jax import lax
from jax.experimental import pallas as pl
from jax.experimental.pallas import tpu as pltpu
from jax.experimental.pallas import tpu_sc as plsc

NC, NS = 2, 16
NW = NC * NS
B = 16384
BPW = B // NW
CH = 128
NCH = BPW // CH
WD, RD = 64, 16
OD = WD + RD + 1
L = 16
G = BPW // L


def _body(weapon_hbm, rank_hbm, level_hbm, wtab2_hbm, rtab2_hbm, out_hbm,
          idx_w, idx_w2, idx_r, lvl, comb, rtab_v, sem):
    wid = lax.axis_index("s") * NC + lax.axis_index("c")
    base = wid * BPW
    rows_out = pl.ds(base, BPW)
    pltpu.sync_copy(weapon_hbm.at[rows_out], idx_w)
    pltpu.sync_copy(rank_hbm.at[rows_out], idx_r)
    pltpu.sync_copy(level_hbm.at[rows_out], lvl)
    pltpu.sync_copy(rtab2_hbm, rtab_v)

    def shift_body(i, _):
        r = pl.ds(i * L, L)
        idx_w2[r] = lax.shift_right_logical(idx_w[r], 1)
        return 0
    lax.fori_loop(0, G, shift_body, 0, unroll=True)

    cps = []
    for j in range(NCH):
        r = pl.ds(j * CH, CH)
        cps.append(pltpu.async_copy(
            wtab2_hbm.at[idx_w2.at[r]], comb.at[r], sem))
    for cp in cps:
        cp.wait()

    lanes = lax.iota(jnp.int32, L)
    col80 = jnp.full((L,), WD + RD, dtype=jnp.int32)

    def group_body(g, _):
        rows16 = lanes + g * L
        sl = pl.ds(g * L, L)
        wi = idx_w[sl]
        half = lax.mul(lax.bitwise_and(wi, 1), WD)
        for c in range(WD):
            v = plsc.load_gather(comb, [rows16, half + c])
            plsc.store_scatter(comb, [rows16, jnp.full((L,), c, jnp.int32)], v)
        ri = idx_r[sl]
        rrow = lax.shift_right_logical(ri, 3)
        rcol = lax.mul(lax.bitwise_and(ri, 7), RD)
        for c in range(RD):
            v = plsc.load_gather(rtab_v, [rrow, rcol + c])
            plsc.store_scatter(
                comb, [rows16, jnp.full((L,), WD + c, jnp.int32)], v)
        plsc.store_scatter(comb, [rows16, col80], lvl[sl])
        return 0

    lax.fori_loop(0, G, group_body, 0)
    pltpu.sync_copy(comb, out_hbm.at[rows_out])


def kernel(weapon, rank, level, weapon_table, rank_table):
    wtab2 = weapon_table.reshape(500000, 128)
    rtab2 = rank_table.reshape(125, 128)
    mesh = plsc.VectorSubcoreMesh(core_axis_name="c", subcore_axis_name="s")
    k = pl.kernel(
        _body,
        out_type=jax.ShapeDtypeStruct((B, 128), jnp.float32),
        mesh=mesh,
        scratch_types=[
            pltpu.VMEM((BPW,), jnp.int32),
            pltpu.VMEM((BPW,), jnp.int32),
            pltpu.VMEM((BPW,), jnp.int32),
            pltpu.VMEM((BPW,), jnp.float32),
            pltpu.VMEM((BPW, 128), jnp.float32),
            pltpu.VMEM((125, 128), jnp.float32),
            pltpu.SemaphoreType.DMA,
        ],
        compiler_params=pltpu.CompilerParams(
            use_tc_tiling_on_sc=True, needs_layout_passes=False),
    )
    out128 = k(weapon, rank, level, wtab2, rtab2)
    return out128[:, :OD]


# speedup vs baseline: 2.6768x; 2.5613x over previous
import jax
import jax.numpy as jnp
from jax import lax
from jax.experimental import pallas as pl
from jax.experimental.pallas import tpu as pltpu
from jax.experimental.pallas import tpu_sc as plsc

NC, NS = 2, 16
NW = NC * NS
B = 16384
BPW = B // NW
WD, RD = 64, 16
OD = WD + RD + 1
L = 16
G = BPW // L
NRING = 4


def _body(weapon_hbm, rank_hbm, level_hbm, wtab_t_hbm, rtab2_hbm, out_hbm,
          idx_w, idx_r, lvl, comb, rtab_v, ring, sems):
    wid = lax.axis_index("s") * NC + lax.axis_index("c")
    base = wid * BPW
    rows_out = pl.ds(base, BPW)
    pltpu.sync_copy(weapon_hbm.at[rows_out], idx_w)
    pltpu.sync_copy(rank_hbm.at[rows_out], idx_r)
    pltpu.sync_copy(level_hbm.at[rows_out], lvl)
    pltpu.sync_copy(rtab2_hbm, rtab_v)

    lanes = lax.iota(jnp.int32, L)
    rows4 = [lanes + L * k for k in range(WD // L)]
    col80 = jnp.full((L,), WD + RD, dtype=jnp.int32)

    def start_fetch(ivec, l, slot):
        i = ivec[l]
        off = pl.multiple_of(lax.shift_right_logical(i, 7) * 128, 128)
        pltpu.async_copy(wtab_t_hbm.at[:, pl.ds(off, 128)],
                         ring.at[slot], sems[slot])

    ivec0 = idx_w[pl.ds(0, L)]
    for l in range(NRING):
        start_fetch(ivec0, l, l)

    def group_body(g, _):
        ivec = idx_w[pl.ds(g * L, L)]
        gn = jnp.minimum(g + 1, G - 1)
        ivec_n = idx_w[pl.ds(gn * L, L)]
        s0 = g * L
        for l in range(L):
            slot = l % NRING
            pltpu.make_async_copy(wtab_t_hbm.at[:, pl.ds(0, 128)],
                                  ring.at[slot], sems[slot]).wait()
            i = ivec[l]
            ccv = jnp.full((L,), lax.bitwise_and(i, 127), jnp.int32)
            for k in range(WD // L):
                v = plsc.load_gather(ring.at[slot], [rows4[k], ccv])
                comb[s0 + l, pl.ds(k * L, L)] = v
            if l < L - NRING:
                start_fetch(ivec, l + NRING, slot)
            else:
                start_fetch(ivec_n, l + NRING - L, slot)
        return 0

    lax.fori_loop(0, G, group_body, 0)
    for l in range(NRING):
        pltpu.make_async_copy(wtab_t_hbm.at[:, pl.ds(0, 128)],
                              ring.at[l], sems[l]).wait()

    def rank_body(g, _):
        rows16 = lanes + g * L
        sl = pl.ds(g * L, L)
        ri = idx_r[sl]
        rrow = lax.shift_right_logical(ri, 3)
        rcol = lax.mul(lax.bitwise_and(ri, 7), RD)
        for c in range(RD):
            v = plsc.load_gather(rtab_v, [rrow, rcol + c])
            plsc.store_scatter(
                comb, [rows16, jnp.full((L,), WD + c, jnp.int32)], v)
        plsc.store_scatter(comb, [rows16, col80], lvl[sl])
        return 0

    lax.fori_loop(0, G, rank_body, 0)
    pltpu.sync_copy(comb, out_hbm.at[rows_out])


def kernel(weapon, rank, level, weapon_table, rank_table):
    wtab_t = weapon_table.T
    rtab2 = rank_table.reshape(125, 128)
    mesh = plsc.VectorSubcoreMesh(core_axis_name="c", subcore_axis_name="s")
    k = pl.kernel(
        _body,
        out_type=jax.ShapeDtypeStruct((B, 128), jnp.float32),
        mesh=mesh,
        scratch_types=[
            pltpu.VMEM((BPW,), jnp.int32),
            pltpu.VMEM((BPW,), jnp.int32),
            pltpu.VMEM((BPW,), jnp.float32),
            pltpu.VMEM((BPW, 128), jnp.float32),
            pltpu.VMEM((125, 128), jnp.float32),
            pltpu.VMEM((NRING, WD, 128), jnp.float32),
            [pltpu.SemaphoreType.DMA] * NRING,
        ],
        compiler_params=pltpu.CompilerParams(
            use_tc_tiling_on_sc=True, needs_layout_passes=False),
    )
    out128 = k(weapon, rank, level, wtab_t, rtab2)
    return out128[:, :OD]


# speedup vs baseline: 3.2050x; 1.1973x over previous
import jax
import jax.numpy as jnp
from jax import lax
from jax.experimental import pallas as pl
from jax.experimental.pallas import tpu as pltpu
from jax.experimental.pallas import tpu_sc as plsc

NC, NS = 2, 16
NW = NC * NS
B = 16384
BPW = B // NW
WD, RD = 64, 16
OD = WD + RD + 1
L = 16
G = BPW // L
NRING = 8


def _body(weapon_hbm, rank_hbm, level_hbm, wtab_t_hbm, rtab2_hbm, out_hbm,
          idx_w, idx_r, lvl, rtab_v, ring, rowbuf, sems):
    wid = lax.axis_index("s") * NC + lax.axis_index("c")
    base = wid * BPW
    rows_out = pl.ds(base, BPW)
    pltpu.sync_copy(weapon_hbm.at[rows_out], idx_w)
    pltpu.sync_copy(rank_hbm.at[rows_out], idx_r)
    pltpu.sync_copy(level_hbm.at[rows_out], lvl)
    pltpu.sync_copy(rtab2_hbm, rtab_v)

    lanes = lax.iota(jnp.int32, L)
    rows4 = [lanes + L * k for k in range(WD // L)]
    col80 = jnp.full((L,), WD + RD, dtype=jnp.int32)

    def start_fetch(ivec, l, slot):
        i = ivec[l]
        off = pl.multiple_of(lax.shift_right_logical(i, 7) * 128, 128)
        pltpu.async_copy(wtab_t_hbm.at[:, pl.ds(off, 128)],
                         ring.at[slot], sems[slot])

    ivec0 = idx_w[pl.ds(0, L)]
    for l in range(NRING):
        start_fetch(ivec0, l, l)

    def group_body(g, _):
        sl = pl.ds(g * L, L)
        ivec = idx_w[sl]
        gn = jnp.minimum(g + 1, G - 1)
        ivec_n = idx_w[pl.ds(gn * L, L)]
        for l in range(L):
            slot = l % NRING
            pltpu.make_async_copy(wtab_t_hbm.at[:, pl.ds(0, 128)],
                                  ring.at[slot], sems[slot]).wait()
            ccv = jnp.full((L,), lax.bitwise_and(ivec[l], 127), jnp.int32)
            for k in range(WD // L):
                rowbuf[l, pl.ds(k * L, L)] = plsc.load_gather(
                    ring.at[slot], [rows4[k], ccv])
            if l < L - NRING:
                start_fetch(ivec, l + NRING, slot)
            else:
                start_fetch(ivec_n, l + NRING - L, slot)
        ri = idx_r[sl]
        rrow = lax.shift_right_logical(ri, 3)
        rcol = lax.mul(lax.bitwise_and(ri, 7), RD)
        for c in range(RD):
            plsc.store_scatter(
                rowbuf, [lanes, jnp.full((L,), WD + c, jnp.int32)],
                plsc.load_gather(rtab_v, [rrow, rcol + c]))
        plsc.store_scatter(rowbuf, [lanes, col80], lvl[sl])
        pltpu.sync_copy(
            rowbuf, out_hbm.at[pl.ds(pl.multiple_of(base + g * L, L), L)])
        return 0

    lax.fori_loop(0, G, group_body, 0)
    for l in range(NRING):
        pltpu.make_async_copy(wtab_t_hbm.at[:, pl.ds(0, 128)],
                              ring.at[l], sems[l]).wait()


def kernel(weapon, rank, level, weapon_table, rank_table):
    wtab_t = weapon_table.T
    rtab2 = rank_table.reshape(125, 128)
    mesh = plsc.VectorSubcoreMesh(core_axis_name="c", subcore_axis_name="s")
    k = pl.kernel(
        _body,
        out_type=jax.ShapeDtypeStruct((B, 128), jnp.float32),
        mesh=mesh,
        scratch_types=[
            pltpu.VMEM((BPW,), jnp.int32),
            pltpu.VMEM((BPW,), jnp.int32),
            pltpu.VMEM((BPW,), jnp.float32),
            pltpu.VMEM((125, 128), jnp.float32),
            pltpu.VMEM((NRING, WD, 128), jnp.float32),
            pltpu.VMEM((L, 128), jnp.float32),
            [pltpu.SemaphoreType.DMA] * NRING,
        ],
        compiler_params=pltpu.CompilerParams(
            use_tc_tiling_on_sc=True, needs_layout_passes=False),
    )
    out128 = k(weapon, rank, level, wtab_t, rtab2)
    return out128[:, :OD]
